# baseline (speedup 1.0000x reference)
import jax
import jax.numpy as jnp
from jax.experimental import pallas as pl
from jax.experimental.pallas import tpu as pltpu

B = 128
IN = 4096
HID = 4096
OUT = 1024
EP = 64
NB = 256

L1_N = HID // NB
L2_I = L1_N
L3_0 = L2_I + 1
L4_0 = L3_0 + HID // NB
L5_0 = L4_0 + HID // NB
STEPS = L5_0 + OUT // NB

_F32 = jnp.float32


def _body(x_ref, wenc1_ref, benc1_ref, wenc2_ref, benc2_ref,
          w1m_ref, w1t_ref, b1_ref, w2_ref, b2_ref, w3_ref, b3_ref,
          out_ref, henc, enc, h1, h2):
    i = pl.program_id(0)

    @pl.when(i < L1_N)
    def _l1():
        acc = jnp.dot(x_ref[...], wenc1_ref[...], preferred_element_type=_F32)
        henc[:, pl.ds(i * NB, NB)] = jnp.maximum(acc + benc1_ref[...], 0.0)

    @pl.when(i == L2_I)
    def _l2():
        enc[...] = (jnp.dot(henc[...], wenc2_ref[...], preferred_element_type=_F32)
                    + benc2_ref[...])

    @pl.when(jnp.logical_and(i >= L3_0, i < L4_0))
    def _l3():
        j = i - L3_0
        acc = (jnp.dot(x_ref[...], w1m_ref[...], preferred_element_type=_F32)
               + jnp.dot(enc[...], w1t_ref[...], preferred_element_type=_F32))
        h1[:, pl.ds(j * NB, NB)] = jnp.maximum(acc + b1_ref[...], 0.0)

    @pl.when(jnp.logical_and(i >= L4_0, i < L5_0))
    def _l4():
        j = i - L4_0
        acc = jnp.dot(h1[...], w2_ref[...], preferred_element_type=_F32)
        h2[:, pl.ds(j * NB, NB)] = jnp.maximum(acc + b2_ref[...], 0.0)

    @pl.when(i >= L5_0)
    def _l5():
        acc = jnp.dot(h2[...], w3_ref[...], preferred_element_type=_F32)
        out_ref[...] = acc + b3_ref[...]


def _j1(i):
    return jnp.clip(i, 0, L1_N - 1)


def _j3(i):
    return jnp.clip(i - L3_0, 0, HID // NB - 1)


def _j4(i):
    return jnp.clip(i - L4_0, 0, HID // NB - 1)


def _j5(i):
    return jnp.clip(i - L5_0, 0, OUT // NB - 1)


def kernel(x, W_enc1, b_enc1, W_enc2, b_enc2, W1, b1, W2, b2, W3, b3):
    benc1 = b_enc1.reshape(1, HID)
    benc2 = b_enc2.reshape(1, EP)
    b1r = b1.reshape(1, HID)
    b2r = b2.reshape(1, HID)
    b3r = b3.reshape(1, OUT)

    in_specs = [
        pl.BlockSpec((B, IN), lambda i: (0, 0)),
        pl.BlockSpec((IN, NB), lambda i: (0, _j1(i))),
        pl.BlockSpec((1, NB), lambda i: (0, _j1(i))),
        pl.BlockSpec((HID, EP), lambda i: (0, 0)),
        pl.BlockSpec((1, EP), lambda i: (0, 0)),
        pl.BlockSpec((IN, NB), lambda i: (0, _j3(i))),
        pl.BlockSpec((EP, NB), lambda i: (IN // EP, _j3(i))),
        pl.BlockSpec((1, NB), lambda i: (0, _j3(i))),
        pl.BlockSpec((HID, NB), lambda i: (0, _j4(i))),
        pl.BlockSpec((1, NB), lambda i: (0, _j4(i))),
        pl.BlockSpec((HID, NB), lambda i: (0, _j5(i))),
        pl.BlockSpec((1, NB), lambda i: (0, _j5(i))),
    ]
    out_spec = pl.BlockSpec((B, NB), lambda i: (0, _j5(i)))

    return pl.pallas_call(
        _body,
        grid=(STEPS,),
        in_specs=in_specs,
        out_specs=out_spec,
        out_shape=jax.ShapeDtypeStruct((B, OUT), _F32),
        scratch_shapes=[
            pltpu.VMEM((B, HID), _F32),
            pltpu.VMEM((B, EP), _F32),
            pltpu.VMEM((B, HID), _F32),
            pltpu.VMEM((B, HID), _F32),
        ],
        compiler_params=pltpu.CompilerParams(
            dimension_semantics=("arbitrary",),
        ),
    )(x, W_enc1, benc1, W_enc2, benc2, W1, W1, b1r, W2, b2r, W3, b3r)
